# Initial kernel scaffold; baseline (speedup 1.0000x reference)
import jax
import jax.numpy as jnp
from jax.experimental import pallas as pl


def kernel(x_product, x_plant, x_group, x_subgroup, x_storage_location, edge_index, type_emb, W0a, b0a, W0b, b0b, W1a, b1a, W1b, b1b, W2a, b2a, W2b, b2b):
    raise NotImplementedError("write your pallas kernel here")


# speedup vs baseline: 3.4030x; 3.4030x over previous
import functools

import jax
import jax.numpy as jnp
from jax import lax
from jax.experimental import pallas as pl
from jax.experimental.pallas import tpu as pltpu
from jax.experimental.pallas import tpu_sc as plsc

_SIZES = (4000, 1500, 1500, 1500, 1500)
_N = 10000
_N_PAD = 10240
_E = 320000
_H = 128
_T = 8
_NW = 32
_CB = 128
_CH = -(-_E // (_NW * _CB))
_E_PAD = _NW * _CB * _CH
_PAD_ROW = _N_PAD - 1
_RPT = _N_PAD // 16


def _make_segsum():
    mesh = plsc.VectorSubcoreMesh(core_axis_name="c", subcore_axis_name="s")

    @functools.partial(
        pl.kernel,
        mesh=mesh,
        out_type=jax.ShapeDtypeStruct((2, _N_PAD, _H), jnp.float32),
        scratch_types=[
            pltpu.VMEM((_CB,), jnp.int32),
            pltpu.VMEM((_CB,), jnp.int32),
            pltpu.VMEM((_CB, _H), jnp.float32),
            pltpu.VMEM_SHARED((_N_PAD, _H), jnp.float32),
            pltpu.SemaphoreType.DMA,
        ],
    )
    def segsum(table, src_idx, dst_idx, zeros, out, src_v, dst_v, rows_v, acc, sem):
        c = lax.axis_index("c")
        s = lax.axis_index("s")
        wid = c * 16 + s
        pltpu.sync_copy(zeros, acc.at[pl.ds(s * _RPT, _RPT)])
        plsc.subcore_barrier()

        def body(ci, carry):
            pltpu.sync_copy(src_idx.at[wid, ci], src_v)
            pltpu.sync_copy(dst_idx.at[wid, ci], dst_v)
            pltpu.async_copy(table.at[src_v], rows_v, sem).wait()
            pltpu.sync_copy(rows_v, acc.at[dst_v], add=True)
            return carry

        lax.fori_loop(0, _CH, body, 0)
        plsc.subcore_barrier()
        pltpu.sync_copy(
            acc.at[pl.ds(s * _RPT, _RPT)],
            out.at[c].at[pl.ds(s * _RPT, _RPT)],
        )

    return segsum


def _proj0_body(x_ref, t_ref, emb_ref, wax_ref, wae_ref, out_ref):
    tb = jnp.dot(emb_ref[...], wae_ref[...], preferred_element_type=jnp.float32)
    out_ref[...] = (
        jnp.dot(x_ref[...], wax_ref[...], preferred_element_type=jnp.float32)
        + jnp.dot(t_ref[...], tb, preferred_element_type=jnp.float32)
    )


def _proj0(x, t_onehot, emb_p, wax, wae):
    bm = 1280
    return pl.pallas_call(
        _proj0_body,
        grid=(_N_PAD // bm,),
        in_specs=[
            pl.BlockSpec((bm, _H), lambda i: (i, 0)),
            pl.BlockSpec((bm, _T), lambda i: (i, 0)),
            pl.BlockSpec((_T, _T), lambda i: (0, 0)),
            pl.BlockSpec((_H, _H), lambda i: (0, 0)),
            pl.BlockSpec((_T, _H), lambda i: (0, 0)),
        ],
        out_specs=pl.BlockSpec((bm, _H), lambda i: (i, 0)),
        out_shape=jax.ShapeDtypeStruct((_N_PAD, _H), jnp.float32),
    )(x, t_onehot, emb_p, wax, wae)


def _mlp_body(g_ref, a0_ref, a1_ref, ba_ref, wb_ref, bb_ref, wan_ref, out_ref):
    t = jnp.maximum(g_ref[...] + a0_ref[...] + a1_ref[...] + ba_ref[...], 0.0)
    h = jnp.maximum(
        jnp.dot(t, wb_ref[...], preferred_element_type=jnp.float32) + bb_ref[...],
        0.0,
    )
    out_ref[...] = jnp.dot(h, wan_ref[...], preferred_element_type=jnp.float32)


def _mlp_last_body(g_ref, a0_ref, a1_ref, ba_ref, wb_ref, bb_ref, out_ref):
    t = jnp.maximum(g_ref[...] + a0_ref[...] + a1_ref[...] + ba_ref[...], 0.0)
    out_ref[...] = jnp.maximum(
        jnp.dot(t, wb_ref[...], preferred_element_type=jnp.float32) + bb_ref[...],
        0.0,
    )


def _mlp(g, a0, a1, ba, wb, bb, wa_next=None):
    bm = 1280
    row = lambda i: (i, 0)
    full = lambda i: (0, 0)
    specs = [
        pl.BlockSpec((bm, _H), row),
        pl.BlockSpec((bm, _H), row),
        pl.BlockSpec((bm, _H), row),
        pl.BlockSpec((1, _H), full),
        pl.BlockSpec((_H, _H), full),
        pl.BlockSpec((1, _H), full),
    ]
    args = [g, a0, a1, ba, wb, bb]
    body = _mlp_last_body
    if wa_next is not None:
        specs.append(pl.BlockSpec((_H, _H), full))
        args.append(wa_next)
        body = _mlp_body
    return pl.pallas_call(
        body,
        grid=(_N_PAD // bm,),
        in_specs=specs,
        out_specs=pl.BlockSpec((bm, _H), row),
        out_shape=jax.ShapeDtypeStruct((_N_PAD, _H), jnp.float32),
    )(*args)


def kernel(x_product, x_plant, x_group, x_subgroup, x_storage_location,
           edge_index, type_emb, W0a, b0a, W0b, b0b, W1a, b1a, W1b, b1b,
           W2a, b2a, W2b, b2b):
    f32 = jnp.float32
    x_all = jnp.concatenate(
        [x_product, x_plant, x_group, x_subgroup, x_storage_location], axis=0)
    x_all = jnp.pad(x_all, ((0, _N_PAD - _N), (0, 0)))

    node_type = jnp.concatenate(
        [jnp.full((n,), i, jnp.int32) for i, n in enumerate(_SIZES)]
        + [jnp.full((_N_PAD - _N,), _T - 1, jnp.int32)])
    t_onehot = (node_type[:, None] == jnp.arange(_T)[None, :]).astype(f32)
    t_onehot = t_onehot.at[_N:].set(0.0)
    emb_p = jnp.pad(type_emb, ((0, _T - 5), (0, 0)))

    pad_e = _E_PAD - _E
    pad_col = jnp.full((pad_e,), _PAD_ROW, jnp.int32)
    src_p = jnp.concatenate([edge_index[0], pad_col]).reshape(_NW, _CH, _CB)
    dst_p = jnp.concatenate([edge_index[1], pad_col]).reshape(_NW, _CH, _CB)
    zeros = jnp.zeros((_RPT, _H), f32)

    seg = _make_segsum()

    g0 = _proj0(x_all, t_onehot, emb_p, W0a[:_H], W0a[_H:])
    agg = seg(g0, src_p, dst_p, zeros)
    g1 = _mlp(g0, agg[0], agg[1], b0a[None], W0b, b0b[None], W1a)
    agg = seg(g1, src_p, dst_p, zeros)
    g2 = _mlp(g1, agg[0], agg[1], b1a[None], W1b, b1b[None], W2a)
    agg = seg(g2, src_p, dst_p, zeros)
    h3 = _mlp(g2, agg[0], agg[1], b2a[None], W2b, b2b[None])
    return h3[:_SIZES[0]]
